# Initial kernel scaffold; baseline (speedup 1.0000x reference)
import jax
import jax.numpy as jnp
from jax.experimental import pallas as pl


def kernel(x, planes_weight):
    raise NotImplementedError("write your pallas kernel here")


# speedup vs baseline: 1.1016x; 1.1016x over previous
import functools

import jax
import jax.numpy as jnp
from jax import lax
from jax.experimental import pallas as pl
from jax.experimental.pallas import tpu as pltpu
from jax.experimental.pallas import tpu_sc as plsc


def _make_gather(V, D, B):
    info = plsc.get_sparse_core_info()
    NC, NS = info.num_cores, info.num_subcores
    NW = NC * NS
    assert B % NW == 0
    b_per_w = B // NW

    CH = 128
    GPB = 8
    ROWS = CH * GPB
    assert b_per_w % ROWS == 0
    n_outer = b_per_w // ROWS

    mesh = plsc.VectorSubcoreMesh(core_axis_name="c", subcore_axis_name="s")

    @functools.partial(
        pl.kernel,
        mesh=mesh,
        compiler_params=pltpu.CompilerParams(use_tc_tiling_on_sc=False),
        out_type=jax.ShapeDtypeStruct((B, D), jnp.float32),
        scratch_types=[
            pltpu.VMEM((b_per_w,), jnp.int32),
            pltpu.VMEM((ROWS, D), jnp.float32),
            pltpu.SemaphoreType.DMA,
        ],
    )
    def gather_kernel(table_hbm, idx_hbm, out_hbm, idx_v, rows_v, gsem):
        wid = lax.axis_index("s") * NC + lax.axis_index("c")
        base = wid * b_per_w
        pltpu.sync_copy(idx_hbm.at[pl.ds(base, b_per_w)], idx_v)

        def body(g, carry):
            off = g * ROWS
            copies = [
                pltpu.async_copy(
                    table_hbm.at[idx_v.at[pl.ds(off + j * CH, CH)]],
                    rows_v.at[pl.ds(j * CH, CH)],
                    gsem,
                )
                for j in range(GPB)
            ]
            for c in copies:
                c.wait()
            pltpu.sync_copy(rows_v, out_hbm.at[pl.ds(base + off, ROWS)])
            return carry

        lax.fori_loop(0, n_outer, body, 0)

    return gather_kernel


def kernel(x, planes_weight):
    V, D = planes_weight.shape
    B = x.size
    idx = x.reshape(B).astype(jnp.int32)
    out = _make_gather(V, D, B)(planes_weight, idx)
    return out.reshape(x.shape + (D,))
